# Initial kernel scaffold; baseline (speedup 1.0000x reference)
import jax
import jax.numpy as jnp
from jax.experimental import pallas as pl


def kernel(var_node_features, con_node_features, edge_index_var, edge_index_con, edge_features_var, edge_features_con, num_nodes_var, num_nodes_con, venc_W1, venc_b1, venc_W2, venc_b2, cenc_W1, cenc_b1, cenc_W2, cenc_b2, ee_W1, ee_b1, ee_W2, ee_b2, mlp_W1, mlp_b1, mlp_W2, mlp_b2, ee_g, ee_bt, mlp_g, mlp_bt, eps, lin1_W, lin1_b, lin2_W, lin2_b, lin3_W, lin3_b, lin4_W, lin4_b):
    raise NotImplementedError("write your pallas kernel here")


# speedup vs baseline: 1.2236x; 1.2236x over previous
import jax
import jax.numpy as jnp
from jax import lax
from jax.experimental import pallas as pl
from jax.experimental.pallas import tpu as pltpu
from jax.experimental.pallas import tpu_sc as plsc

_NV = 50000
_E = 800000
_H = 64

_NBLK = 2000
_EBLK = 4000

_SC_HALF = 25000
_HALF2 = 12500
_ACC_ROWS = 12672
_RPT = 792
_CHUNK = 80
_EPSC = 50000
_NCHUNK = 625


def _enc_body(x_ref, w1_ref, b1_ref, w2_ref, b2_ref, o_ref):
    h = jnp.maximum(
        jnp.dot(x_ref[...], w1_ref[...], preferred_element_type=jnp.float32)
        + b1_ref[...], 0.0)
    h2 = (jnp.dot(h, w2_ref[...], preferred_element_type=jnp.float32)
          + b2_ref[...])
    o_ref[...] = jnp.concatenate(
        [h2, jnp.zeros((h2.shape[0], _H), jnp.float32)], axis=1)


def _encoder(x, w1, b1, w2, b2):
    n = x.shape[0]
    return pl.pallas_call(
        _enc_body,
        grid=(n // _NBLK,),
        in_specs=[
            pl.BlockSpec((_NBLK, x.shape[1]), lambda i: (i, 0)),
            pl.BlockSpec(w1.shape, lambda i: (0, 0)),
            pl.BlockSpec(b1.shape, lambda i: (0, 0)),
            pl.BlockSpec(w2.shape, lambda i: (0, 0)),
            pl.BlockSpec(b2.shape, lambda i: (0, 0)),
        ],
        out_specs=pl.BlockSpec((_NBLK, 2 * _H), lambda i: (i, 0)),
        out_shape=jax.ShapeDtypeStruct((n, 2 * _H), jnp.float32),
    )(x, w1, b1, w2, b2)


def _edge_enc_body(a_ref, w1_ref, b1_ref, w2_ref, b2_ref, h_ref, st_ref):
    h1 = jnp.maximum(a_ref[...] * w1_ref[...] + b1_ref[...], 0.0)
    h2 = jnp.maximum(
        jnp.dot(h1, w2_ref[...], preferred_element_type=jnp.float32)
        + b2_ref[...], 0.0)
    h_ref[...] = jnp.concatenate(
        [jnp.zeros((h2.shape[0], _H), jnp.float32), h2], axis=1)
    s1 = jnp.sum(h2, axis=0, keepdims=True)
    s2 = jnp.sum(h2 * h2, axis=0, keepdims=True)

    @pl.when(pl.program_id(0) == 0)
    def _():
        st_ref[...] = jnp.zeros_like(st_ref)

    st_ref[...] += jnp.concatenate(
        [s1, s2, jnp.zeros((6, _H), jnp.float32)], axis=0)


def _edge_enc(a, w1, b1, w2, b2):
    return pl.pallas_call(
        _edge_enc_body,
        grid=(_E // _EBLK,),
        in_specs=[
            pl.BlockSpec((_EBLK, 1), lambda i: (i, 0)),
            pl.BlockSpec(w1.shape, lambda i: (0, 0)),
            pl.BlockSpec(b1.shape, lambda i: (0, 0)),
            pl.BlockSpec(w2.shape, lambda i: (0, 0)),
            pl.BlockSpec(b2.shape, lambda i: (0, 0)),
        ],
        out_specs=[
            pl.BlockSpec((_EBLK, 2 * _H), lambda i: (i, 0)),
            pl.BlockSpec((8, _H), lambda i: (0, 0)),
        ],
        out_shape=[
            jax.ShapeDtypeStruct((_E, 2 * _H), jnp.float32),
            jax.ShapeDtypeStruct((8, _H), jnp.float32),
        ],
    )(a, w1, b1, w2, b2)


def _node_mlp_body(tmp_ref, tgt_ref, epsr_ref, w1_ref, b1_ref, w2_ref, b2_ref,
                   h_ref, st_ref):
    h0 = tgt_ref[...] * epsr_ref[...] + tmp_ref[...]
    h1 = jnp.maximum(
        jnp.dot(h0, w1_ref[...], preferred_element_type=jnp.float32)
        + b1_ref[...], 0.0)
    h2 = jnp.maximum(
        jnp.dot(h1, w2_ref[...], preferred_element_type=jnp.float32)
        + b2_ref[...], 0.0)
    h_ref[...] = h2
    s1 = jnp.sum(h2, axis=0, keepdims=True)
    s2 = jnp.sum(h2 * h2, axis=0, keepdims=True)

    @pl.when(pl.program_id(0) == 0)
    def _():
        st_ref[...] = jnp.zeros_like(st_ref)

    st_ref[...] += jnp.concatenate(
        [s1, s2, jnp.zeros((6, _H), jnp.float32)], axis=0)


def _node_mlp(tmp, tgt, epsr, w1, b1, w2, b2):
    return pl.pallas_call(
        _node_mlp_body,
        grid=(_NV // _NBLK,),
        in_specs=[
            pl.BlockSpec((_NBLK, _H), lambda i: (i, 0)),
            pl.BlockSpec((_NBLK, _H), lambda i: (i, 0)),
            pl.BlockSpec((1, 1), lambda i: (0, 0)),
            pl.BlockSpec(w1.shape, lambda i: (0, 0)),
            pl.BlockSpec(b1.shape, lambda i: (0, 0)),
            pl.BlockSpec(w2.shape, lambda i: (0, 0)),
            pl.BlockSpec(b2.shape, lambda i: (0, 0)),
        ],
        out_specs=[
            pl.BlockSpec((_NBLK, _H), lambda i: (i, 0)),
            pl.BlockSpec((8, _H), lambda i: (0, 0)),
        ],
        out_shape=[
            jax.ShapeDtypeStruct((_NV, _H), jnp.float32),
            jax.ShapeDtypeStruct((8, _H), jnp.float32),
        ],
    )(tmp, tgt, epsr, w1, b1, w2, b2)


def _affine_relu_body(h_ref, sc_ref, sh_ref, o_ref):
    y = jnp.maximum(h_ref[...] * sc_ref[...] + sh_ref[...], 0.0)
    o_ref[...] = jnp.concatenate(
        [y, jnp.zeros((y.shape[0], _H), jnp.float32)], axis=1)


def _affine_relu(h, scl, shf):
    return pl.pallas_call(
        _affine_relu_body,
        grid=(_NV // _NBLK,),
        in_specs=[
            pl.BlockSpec((_NBLK, _H), lambda i: (i, 0)),
            pl.BlockSpec((1, _H), lambda i: (0, 0)),
            pl.BlockSpec((1, _H), lambda i: (0, 0)),
        ],
        out_specs=pl.BlockSpec((_NBLK, 2 * _H), lambda i: (i, 0)),
        out_shape=jax.ShapeDtypeStruct((_NV, 2 * _H), jnp.float32),
    )(h, scl, shf)


def _head_body(x_ref, w1_ref, b1_ref, w2_ref, b2_ref, w3_ref, b3_ref,
               w4_ref, b4_ref, o_ref):
    h = jnp.maximum(
        jnp.dot(x_ref[...], w1_ref[...], preferred_element_type=jnp.float32)
        + b1_ref[...], 0.0)
    h = jnp.maximum(
        jnp.dot(h, w2_ref[...], preferred_element_type=jnp.float32)
        + b2_ref[...], 0.0)
    h = jnp.maximum(
        jnp.dot(h, w3_ref[...], preferred_element_type=jnp.float32)
        + b3_ref[...], 0.0)
    z = (jnp.dot(h, w4_ref[...], preferred_element_type=jnp.float32)
         + b4_ref[...])
    m = jnp.max(z, axis=-1, keepdims=True)
    o_ref[...] = z - (m + jnp.log(jnp.sum(jnp.exp(z - m), axis=-1,
                                          keepdims=True)))


def _head(x, w1, b1, w2, b2, w3, b3, w4, b4):
    return pl.pallas_call(
        _head_body,
        grid=(_NV // _NBLK,),
        in_specs=[
            pl.BlockSpec((_NBLK, 5 * _H), lambda i: (i, 0)),
            pl.BlockSpec(w1.shape, lambda i: (0, 0)),
            pl.BlockSpec(b1.shape, lambda i: (0, 0)),
            pl.BlockSpec(w2.shape, lambda i: (0, 0)),
            pl.BlockSpec(b2.shape, lambda i: (0, 0)),
            pl.BlockSpec(w3.shape, lambda i: (0, 0)),
            pl.BlockSpec(b3.shape, lambda i: (0, 0)),
            pl.BlockSpec(w4.shape, lambda i: (0, 0)),
            pl.BlockSpec(b4.shape, lambda i: (0, 0)),
        ],
        out_specs=pl.BlockSpec((_NBLK, 2), lambda i: (i, 0)),
        out_shape=jax.ShapeDtypeStruct((_NV, 2), jnp.float32),
    )(x, w1, b1, w2, b2, w3, b3, w4, b4)


def _sc_msg_body(x_hbm, src_hbm, dst_hbm, e_hbm, sc_hbm, sh_hbm, z_hbm,
                 out_hbm, acc, src_v, dst_v, dstl_v, dstr_v, xrows, mrows,
                 scale_v, shift_v, sem):
    cid = lax.axis_index("c")
    sid = lax.axis_index("s")
    lo = cid * _SC_HALF
    pltpu.sync_copy(z_hbm, acc.at[pl.ds(sid * _RPT, _RPT)])
    pltpu.sync_copy(sc_hbm, scale_v)
    pltpu.sync_copy(sh_hbm, shift_v)
    plsc.subcore_barrier()
    scs = [scale_v[pl.ds(16 * k, 16)] for k in range(4)]
    shs = [shift_v[pl.ds(16 * k, 16)] for k in range(4)]
    iota16 = lax.iota(jnp.int32, 16)
    ebase = sid * _EPSC

    def chunk(k, carry):
        base = ebase + k * _CHUNK
        pltpu.sync_copy(src_hbm.at[pl.ds(base, _CHUNK)], src_v)
        pltpu.sync_copy(dst_hbm.at[pl.ds(base, _CHUNK)], dst_v)
        pltpu.async_copy(x_hbm.at[src_v], xrows, sem).wait()
        pltpu.sync_copy(e_hbm.at[pl.ds(base, _CHUNK)], mrows)

        def row4(r4, c2):
            r = r4 * 4
            for rr in range(4):
                for cc in range(4):
                    xv = xrows[r + rr, pl.ds(16 * cc, 16)]
                    ev = mrows[r + rr, pl.ds(_H + 16 * cc, 16)]
                    m = jnp.maximum(xv + (ev * scs[cc] + shs[cc]), 0.0)
                    xrows[r + rr, pl.ds(16 * cc, 16)] = m
                    mrows[r + rr, pl.ds(_H + 16 * cc, 16)] = m
            return c2

        lax.fori_loop(0, _CHUNK // 4, row4, 0)

        def clamp(j, c2):
            d = dst_v[pl.ds(16 * j, 16)]
            dl = d - lo
            ok = (dl >= 0) & (dl < _SC_HALF)
            disc = _HALF2 + 16 * j + iota16
            dstl_v[pl.ds(16 * j, 16)] = jnp.where(
                ok & (dl < _HALF2), dl, disc)
            dstr_v[pl.ds(16 * j, 16)] = jnp.where(
                ok & (dl >= _HALF2), dl - _HALF2, disc)
            return c2

        lax.fori_loop(0, _CHUNK // 16, clamp, 0)
        pltpu.sync_copy(xrows, acc.at[dstl_v], add=True)
        pltpu.sync_copy(mrows, acc.at[dstr_v], add=True)
        return carry

    lax.fori_loop(0, _NCHUNK, chunk, 0)
    plsc.subcore_barrier()
    pltpu.sync_copy(acc.at[pl.ds(sid * _RPT, _RPT)],
                    out_hbm.at[pl.ds(cid * _ACC_ROWS + sid * _RPT, _RPT)])


def _sc_message(x, src, dst, e, scl, shf, zrows):
    mesh = plsc.VectorSubcoreMesh(core_axis_name="c", subcore_axis_name="s")
    fn = pl.kernel(
        _sc_msg_body,
        mesh=mesh,
        out_type=jax.ShapeDtypeStruct((2 * _ACC_ROWS, 2 * _H), jnp.float32),
        scratch_types=[
            pltpu.VMEM_SHARED((_ACC_ROWS, 2 * _H), jnp.float32),
            pltpu.VMEM((_CHUNK,), jnp.int32),
            pltpu.VMEM((_CHUNK,), jnp.int32),
            pltpu.VMEM((_CHUNK,), jnp.int32),
            pltpu.VMEM((_CHUNK,), jnp.int32),
            pltpu.VMEM((_CHUNK, 2 * _H), jnp.float32),
            pltpu.VMEM((_CHUNK, 2 * _H), jnp.float32),
            pltpu.VMEM((_H,), jnp.float32),
            pltpu.VMEM((_H,), jnp.float32),
            pltpu.SemaphoreType.DMA,
        ],
    )
    return fn(x, src, dst, e, scl, shf, zrows)


def _bn_affine(st, n, g, bt):
    m = st[0] / n
    var = st[1] / n - m * m
    scl = g * lax.rsqrt(var + 1e-5)
    return scl, bt - m * scl


def _round(i, src_pad, tgt, ei, ef, ee_W1, ee_b1, ee_W2, ee_b2,
           mlp_W1, mlp_b1, mlp_W2, mlp_b2, ee_g, ee_bt, mlp_g, mlp_bt,
           eps, zrows):
    h2e, ste = _edge_enc(ef, ee_W1[i], ee_b1[i].reshape(1, _H),
                         ee_W2[i], ee_b2[i].reshape(1, _H))
    scl_e, shf_e = _bn_affine(ste, float(_E), ee_g[i], ee_bt[i])
    tmp_pad = _sc_message(src_pad, ei[0], ei[1], h2e, scl_e, shf_e, zrows)
    b0 = tmp_pad[0:_HALF2]
    b1 = tmp_pad[_ACC_ROWS:_ACC_ROWS + _HALF2]
    tmp = jnp.concatenate(
        [b0[:, :_H], b0[:, _H:], b1[:, :_H], b1[:, _H:]], axis=0)
    epsr = (1.0 + eps[i]).reshape(1, 1)
    h2n, stn = _node_mlp(tmp, tgt, epsr, mlp_W1[i], mlp_b1[i].reshape(1, _H),
                         mlp_W2[i], mlp_b2[i].reshape(1, _H))
    scl_n, shf_n = _bn_affine(stn, float(_NV), mlp_g[i], mlp_bt[i])
    return _affine_relu(h2n, scl_n.reshape(1, _H), shf_n.reshape(1, _H))


def kernel(var_node_features, con_node_features, edge_index_var,
           edge_index_con, edge_features_var, edge_features_con,
           num_nodes_var, num_nodes_con, venc_W1, venc_b1, venc_W2, venc_b2,
           cenc_W1, cenc_b1, cenc_W2, cenc_b2, ee_W1, ee_b1, ee_W2, ee_b2,
           mlp_W1, mlp_b1, mlp_W2, mlp_b2, ee_g, ee_bt, mlp_g, mlp_bt, eps,
           lin1_W, lin1_b, lin2_W, lin2_b, lin3_W, lin3_b, lin4_W, lin4_b):
    vp = _encoder(var_node_features, venc_W1, venc_b1.reshape(1, _H),
                  venc_W2, venc_b2.reshape(1, _H))
    cp = _encoder(con_node_features, cenc_W1, cenc_b1.reshape(1, _H),
                  cenc_W2, cenc_b2.reshape(1, _H))
    zrows = jnp.zeros((_RPT, 2 * _H), jnp.float32)
    v64 = vp[:, :_H]
    c64 = cp[:, :_H]
    vs = [v64]
    for t in range(4):
        cp = _round(2 * t, vp, c64, edge_index_var, edge_features_var,
                    ee_W1, ee_b1, ee_W2, ee_b2, mlp_W1, mlp_b1, mlp_W2,
                    mlp_b2, ee_g, ee_bt, mlp_g, mlp_bt, eps, zrows)
        c64 = cp[:, :_H]
        vp = _round(2 * t + 1, cp, v64, edge_index_con, edge_features_con,
                    ee_W1, ee_b1, ee_W2, ee_b2, mlp_W1, mlp_b1, mlp_W2,
                    mlp_b2, ee_g, ee_bt, mlp_g, mlp_bt, eps, zrows)
        v64 = vp[:, :_H]
        vs.append(v64)
    x = jnp.concatenate(vs, axis=-1)
    return _head(x, lin1_W, lin1_b.reshape(1, _H), lin2_W,
                 lin2_b.reshape(1, _H), lin3_W, lin3_b.reshape(1, _H),
                 lin4_W, lin4_b.reshape(1, 2))


# speedup vs baseline: 1.3103x; 1.0708x over previous
import jax
import jax.numpy as jnp
from jax import lax
from jax.experimental import pallas as pl
from jax.experimental.pallas import tpu as pltpu
from jax.experimental.pallas import tpu_sc as plsc

_NV = 50000
_E = 800000
_H = 64

_NBLK = 2000
_EBLK = 4000

_SC_HALF = 25000
_HALF2 = 12500
_ACC_ROWS = 12672
_RPT = 792
_CHUNK = 80
_EPSC = 50000
_NCHUNK = 625


def _enc_body(x_ref, w1_ref, b1_ref, w2_ref, b2_ref, o_ref):
    h = jnp.maximum(
        jnp.dot(x_ref[...], w1_ref[...], preferred_element_type=jnp.float32)
        + b1_ref[...], 0.0)
    h2 = (jnp.dot(h, w2_ref[...], preferred_element_type=jnp.float32)
          + b2_ref[...])
    o_ref[...] = jnp.concatenate(
        [h2, jnp.zeros((h2.shape[0], _H), jnp.float32)], axis=1)


def _encoder(x, w1, b1, w2, b2):
    n = x.shape[0]
    return pl.pallas_call(
        _enc_body,
        grid=(n // _NBLK,),
        in_specs=[
            pl.BlockSpec((_NBLK, x.shape[1]), lambda i: (i, 0)),
            pl.BlockSpec(w1.shape, lambda i: (0, 0)),
            pl.BlockSpec(b1.shape, lambda i: (0, 0)),
            pl.BlockSpec(w2.shape, lambda i: (0, 0)),
            pl.BlockSpec(b2.shape, lambda i: (0, 0)),
        ],
        out_specs=pl.BlockSpec((_NBLK, 2 * _H), lambda i: (i, 0)),
        out_shape=jax.ShapeDtypeStruct((n, 2 * _H), jnp.float32),
    )(x, w1, b1, w2, b2)


def _edge_enc_body(a_ref, w1_ref, b1_ref, w2_ref, b2_ref, h_ref, st_ref):
    h1 = jnp.maximum(a_ref[...] * w1_ref[...] + b1_ref[...], 0.0)
    h2 = jnp.maximum(
        jnp.dot(h1, w2_ref[...], preferred_element_type=jnp.float32)
        + b2_ref[...], 0.0)
    h_ref[...] = jnp.concatenate(
        [jnp.zeros((h2.shape[0], _H), jnp.float32), h2], axis=1)
    s1 = jnp.sum(h2, axis=0, keepdims=True)
    s2 = jnp.sum(h2 * h2, axis=0, keepdims=True)

    @pl.when(pl.program_id(0) == 0)
    def _():
        st_ref[...] = jnp.zeros_like(st_ref)

    st_ref[...] += jnp.concatenate(
        [s1, s2, jnp.zeros((6, _H), jnp.float32)], axis=0)


def _edge_enc(a, w1, b1, w2, b2):
    return pl.pallas_call(
        _edge_enc_body,
        grid=(_E // _EBLK,),
        in_specs=[
            pl.BlockSpec((_EBLK, 1), lambda i: (i, 0)),
            pl.BlockSpec(w1.shape, lambda i: (0, 0)),
            pl.BlockSpec(b1.shape, lambda i: (0, 0)),
            pl.BlockSpec(w2.shape, lambda i: (0, 0)),
            pl.BlockSpec(b2.shape, lambda i: (0, 0)),
        ],
        out_specs=[
            pl.BlockSpec((_EBLK, 2 * _H), lambda i: (i, 0)),
            pl.BlockSpec((8, _H), lambda i: (0, 0)),
        ],
        out_shape=[
            jax.ShapeDtypeStruct((_E, 2 * _H), jnp.float32),
            jax.ShapeDtypeStruct((8, _H), jnp.float32),
        ],
    )(a, w1, b1, w2, b2)


def _node_mlp_body(tmp_ref, tgt_ref, epsr_ref, w1_ref, b1_ref, w2_ref, b2_ref,
                   h_ref, st_ref):
    h0 = tgt_ref[...] * epsr_ref[...] + tmp_ref[...]
    h1 = jnp.maximum(
        jnp.dot(h0, w1_ref[...], preferred_element_type=jnp.float32)
        + b1_ref[...], 0.0)
    h2 = jnp.maximum(
        jnp.dot(h1, w2_ref[...], preferred_element_type=jnp.float32)
        + b2_ref[...], 0.0)
    h_ref[...] = h2
    s1 = jnp.sum(h2, axis=0, keepdims=True)
    s2 = jnp.sum(h2 * h2, axis=0, keepdims=True)

    @pl.when(pl.program_id(0) == 0)
    def _():
        st_ref[...] = jnp.zeros_like(st_ref)

    st_ref[...] += jnp.concatenate(
        [s1, s2, jnp.zeros((6, _H), jnp.float32)], axis=0)


def _node_mlp(tmp, tgt, epsr, w1, b1, w2, b2):
    return pl.pallas_call(
        _node_mlp_body,
        grid=(_NV // _NBLK,),
        in_specs=[
            pl.BlockSpec((_NBLK, _H), lambda i: (i, 0)),
            pl.BlockSpec((_NBLK, _H), lambda i: (i, 0)),
            pl.BlockSpec((1, 1), lambda i: (0, 0)),
            pl.BlockSpec(w1.shape, lambda i: (0, 0)),
            pl.BlockSpec(b1.shape, lambda i: (0, 0)),
            pl.BlockSpec(w2.shape, lambda i: (0, 0)),
            pl.BlockSpec(b2.shape, lambda i: (0, 0)),
        ],
        out_specs=[
            pl.BlockSpec((_NBLK, _H), lambda i: (i, 0)),
            pl.BlockSpec((8, _H), lambda i: (0, 0)),
        ],
        out_shape=[
            jax.ShapeDtypeStruct((_NV, _H), jnp.float32),
            jax.ShapeDtypeStruct((8, _H), jnp.float32),
        ],
    )(tmp, tgt, epsr, w1, b1, w2, b2)


def _affine_relu_body(h_ref, sc_ref, sh_ref, o_ref):
    y = jnp.maximum(h_ref[...] * sc_ref[...] + sh_ref[...], 0.0)
    o_ref[...] = jnp.concatenate(
        [y, jnp.zeros((y.shape[0], _H), jnp.float32)], axis=1)


def _affine_relu(h, scl, shf):
    return pl.pallas_call(
        _affine_relu_body,
        grid=(_NV // _NBLK,),
        in_specs=[
            pl.BlockSpec((_NBLK, _H), lambda i: (i, 0)),
            pl.BlockSpec((1, _H), lambda i: (0, 0)),
            pl.BlockSpec((1, _H), lambda i: (0, 0)),
        ],
        out_specs=pl.BlockSpec((_NBLK, 2 * _H), lambda i: (i, 0)),
        out_shape=jax.ShapeDtypeStruct((_NV, 2 * _H), jnp.float32),
    )(h, scl, shf)


def _head_body(x_ref, w1_ref, b1_ref, w2_ref, b2_ref, w3_ref, b3_ref,
               w4_ref, b4_ref, o_ref):
    h = jnp.maximum(
        jnp.dot(x_ref[...], w1_ref[...], preferred_element_type=jnp.float32)
        + b1_ref[...], 0.0)
    h = jnp.maximum(
        jnp.dot(h, w2_ref[...], preferred_element_type=jnp.float32)
        + b2_ref[...], 0.0)
    h = jnp.maximum(
        jnp.dot(h, w3_ref[...], preferred_element_type=jnp.float32)
        + b3_ref[...], 0.0)
    z = (jnp.dot(h, w4_ref[...], preferred_element_type=jnp.float32)
         + b4_ref[...])
    m = jnp.max(z, axis=-1, keepdims=True)
    o_ref[...] = z - (m + jnp.log(jnp.sum(jnp.exp(z - m), axis=-1,
                                          keepdims=True)))


def _head(x, w1, b1, w2, b2, w3, b3, w4, b4):
    return pl.pallas_call(
        _head_body,
        grid=(_NV // _NBLK,),
        in_specs=[
            pl.BlockSpec((_NBLK, 5 * _H), lambda i: (i, 0)),
            pl.BlockSpec(w1.shape, lambda i: (0, 0)),
            pl.BlockSpec(b1.shape, lambda i: (0, 0)),
            pl.BlockSpec(w2.shape, lambda i: (0, 0)),
            pl.BlockSpec(b2.shape, lambda i: (0, 0)),
            pl.BlockSpec(w3.shape, lambda i: (0, 0)),
            pl.BlockSpec(b3.shape, lambda i: (0, 0)),
            pl.BlockSpec(w4.shape, lambda i: (0, 0)),
            pl.BlockSpec(b4.shape, lambda i: (0, 0)),
        ],
        out_specs=pl.BlockSpec((_NBLK, 2), lambda i: (i, 0)),
        out_shape=jax.ShapeDtypeStruct((_NV, 2), jnp.float32),
    )(x, w1, b1, w2, b2, w3, b3, w4, b4)


def _sc_msg_body(x_hbm, src_hbm, dst_hbm, e_hbm, sc_hbm, sh_hbm, z_hbm,
                 out_hbm, acc, src_v, dst_v, dstm_v, xrows, mrows,
                 scale_v, shift_v, sem):
    cid = lax.axis_index("c")
    sid = lax.axis_index("s")
    lo = cid * _SC_HALF
    pltpu.sync_copy(z_hbm, acc.at[pl.ds(sid * _RPT, _RPT)])
    pltpu.sync_copy(sc_hbm, scale_v)
    pltpu.sync_copy(sh_hbm, shift_v)
    plsc.subcore_barrier()
    scs = [scale_v[pl.ds(16 * k, 16)] for k in range(4)]
    shs = [shift_v[pl.ds(16 * k, 16)] for k in range(4)]
    iota16 = lax.iota(jnp.int32, 16)
    ebase = sid * _EPSC

    def chunk(k, carry):
        base = ebase + k * _CHUNK
        pltpu.sync_copy(src_hbm.at[pl.ds(base, _CHUNK)], src_v)
        pltpu.sync_copy(dst_hbm.at[pl.ds(base, _CHUNK)], dst_v)
        pltpu.async_copy(x_hbm.at[src_v], xrows, sem).wait()
        pltpu.sync_copy(e_hbm.at[pl.ds(base, _CHUNK)], mrows)
        for j in range(_CHUNK // 16):
            d16 = dst_v[pl.ds(16 * j, 16)]
            dl = d16 - lo
            ok = (dl >= 0) & (dl < _SC_HALF)
            is_l = ok & (dl < _HALF2)
            is_r = ok & (dl >= _HALF2)
            disc = _HALF2 + 16 * j + iota16
            dstm_v[pl.ds(16 * j, 16)] = jnp.where(
                is_l, dl, jnp.where(is_r, dl - _HALF2, disc))
            ml16 = jnp.where(is_l, 1.0, 0.0)
            mr16 = jnp.where(is_r, 1.0, 0.0)
            for rr in range(16):
                r = 16 * j + rr
                mls = jnp.broadcast_to(lax.slice(ml16, (rr,), (rr + 1,)), (16,))
                mrs = jnp.broadcast_to(lax.slice(mr16, (rr,), (rr + 1,)), (16,))
                for cc in range(4):
                    xv = xrows[r, pl.ds(16 * cc, 16)]
                    ev = mrows[r, pl.ds(_H + 16 * cc, 16)]
                    m = jnp.maximum(xv + (ev * scs[cc] + shs[cc]), 0.0)
                    mrows[r, pl.ds(16 * cc, 16)] = m * mls
                    mrows[r, pl.ds(_H + 16 * cc, 16)] = m * mrs
        pltpu.sync_copy(mrows, acc.at[dstm_v], add=True)
        return carry

    lax.fori_loop(0, _NCHUNK, chunk, 0)
    plsc.subcore_barrier()
    pltpu.sync_copy(acc.at[pl.ds(sid * _RPT, _RPT)],
                    out_hbm.at[pl.ds(cid * _ACC_ROWS + sid * _RPT, _RPT)])


def _sc_message(x, src, dst, e, scl, shf, zrows):
    mesh = plsc.VectorSubcoreMesh(core_axis_name="c", subcore_axis_name="s")
    fn = pl.kernel(
        _sc_msg_body,
        mesh=mesh,
        out_type=jax.ShapeDtypeStruct((2 * _ACC_ROWS, 2 * _H), jnp.float32),
        scratch_types=[
            pltpu.VMEM_SHARED((_ACC_ROWS, 2 * _H), jnp.float32),
            pltpu.VMEM((_CHUNK,), jnp.int32),
            pltpu.VMEM((_CHUNK,), jnp.int32),
            pltpu.VMEM((_CHUNK,), jnp.int32),
            pltpu.VMEM((_CHUNK, 2 * _H), jnp.float32),
            pltpu.VMEM((_CHUNK, 2 * _H), jnp.float32),
            pltpu.VMEM((_H,), jnp.float32),
            pltpu.VMEM((_H,), jnp.float32),
            pltpu.SemaphoreType.DMA,
        ],
    )
    return fn(x, src, dst, e, scl, shf, zrows)


def _bn_affine(st, n, g, bt):
    m = st[0] / n
    var = st[1] / n - m * m
    scl = g * lax.rsqrt(var + 1e-5)
    return scl, bt - m * scl


def _round(i, src_pad, tgt, ei, ef, ee_W1, ee_b1, ee_W2, ee_b2,
           mlp_W1, mlp_b1, mlp_W2, mlp_b2, ee_g, ee_bt, mlp_g, mlp_bt,
           eps, zrows):
    h2e, ste = _edge_enc(ef, ee_W1[i], ee_b1[i].reshape(1, _H),
                         ee_W2[i], ee_b2[i].reshape(1, _H))
    scl_e, shf_e = _bn_affine(ste, float(_E), ee_g[i], ee_bt[i])
    tmp_pad = _sc_message(src_pad, ei[0], ei[1], h2e, scl_e, shf_e, zrows)
    b0 = tmp_pad[0:_HALF2]
    b1 = tmp_pad[_ACC_ROWS:_ACC_ROWS + _HALF2]
    tmp = jnp.concatenate(
        [b0[:, :_H], b0[:, _H:], b1[:, :_H], b1[:, _H:]], axis=0)
    epsr = (1.0 + eps[i]).reshape(1, 1)
    h2n, stn = _node_mlp(tmp, tgt, epsr, mlp_W1[i], mlp_b1[i].reshape(1, _H),
                         mlp_W2[i], mlp_b2[i].reshape(1, _H))
    scl_n, shf_n = _bn_affine(stn, float(_NV), mlp_g[i], mlp_bt[i])
    return _affine_relu(h2n, scl_n.reshape(1, _H), shf_n.reshape(1, _H))


def kernel(var_node_features, con_node_features, edge_index_var,
           edge_index_con, edge_features_var, edge_features_con,
           num_nodes_var, num_nodes_con, venc_W1, venc_b1, venc_W2, venc_b2,
           cenc_W1, cenc_b1, cenc_W2, cenc_b2, ee_W1, ee_b1, ee_W2, ee_b2,
           mlp_W1, mlp_b1, mlp_W2, mlp_b2, ee_g, ee_bt, mlp_g, mlp_bt, eps,
           lin1_W, lin1_b, lin2_W, lin2_b, lin3_W, lin3_b, lin4_W, lin4_b):
    vp = _encoder(var_node_features, venc_W1, venc_b1.reshape(1, _H),
                  venc_W2, venc_b2.reshape(1, _H))
    cp = _encoder(con_node_features, cenc_W1, cenc_b1.reshape(1, _H),
                  cenc_W2, cenc_b2.reshape(1, _H))
    zrows = jnp.zeros((_RPT, 2 * _H), jnp.float32)
    v64 = vp[:, :_H]
    c64 = cp[:, :_H]
    vs = [v64]
    for t in range(4):
        cp = _round(2 * t, vp, c64, edge_index_var, edge_features_var,
                    ee_W1, ee_b1, ee_W2, ee_b2, mlp_W1, mlp_b1, mlp_W2,
                    mlp_b2, ee_g, ee_bt, mlp_g, mlp_bt, eps, zrows)
        c64 = cp[:, :_H]
        vp = _round(2 * t + 1, cp, v64, edge_index_con, edge_features_con,
                    ee_W1, ee_b1, ee_W2, ee_b2, mlp_W1, mlp_b1, mlp_W2,
                    mlp_b2, ee_g, ee_bt, mlp_g, mlp_bt, eps, zrows)
        v64 = vp[:, :_H]
        vs.append(v64)
    x = jnp.concatenate(vs, axis=-1)
    return _head(x, lin1_W, lin1_b.reshape(1, _H), lin2_W,
                 lin2_b.reshape(1, _H), lin3_W, lin3_b.reshape(1, _H),
                 lin4_W, lin4_b.reshape(1, 2))
